# speedup vs baseline: 87.1279x; 87.1279x over previous
import jax
import jax.numpy as jnp
from jax.experimental import pallas as pl
from jax.experimental.pallas import tpu as pltpu

_N = 5000
_B = 512
_NB = 10
_NP = _B * _NB
_IOU_T = 0.3
_SCORE_T = 0.05
_TOPK = 15


def _row_coords(raw_s_ref, i):
    blk = raw_s_ref[i * _B:(i + 1) * _B, :]
    x1 = blk[:, 0:1] * 1024.0
    y1 = blk[:, 1:2] * 1024.0
    w = blk[:, 2:3] * 200.0 + 1.0
    h = blk[:, 3:4] * 200.0 + 1.0
    x2 = x1 + w
    y2 = y1 + h
    area = (x2 - x1) * (y2 - y1)
    return x1, y1, x2, y2, area


def _col_coords(raw_t_ref, j):
    blk = raw_t_ref[:, j * _B:(j + 1) * _B]
    x1 = blk[0:1, :] * 1024.0
    y1 = blk[1:2, :] * 1024.0
    w = blk[2:3, :] * 200.0 + 1.0
    h = blk[3:4, :] * 200.0 + 1.0
    x2 = x1 + w
    y2 = y1 + h
    area = (x2 - x1) * (y2 - y1)
    return x1, y1, x2, y2, area


def _sup_mat(row, col):
    rx1, ry1, rx2, ry2, ra = row
    cx1, cy1, cx2, cy2, ca = col
    iw = jnp.maximum(jnp.minimum(rx2, cx2) - jnp.maximum(rx1, cx1), 0.0)
    ih = jnp.maximum(jnp.minimum(ry2, cy2) - jnp.maximum(ry1, cy1), 0.0)
    inter = iw * ih
    union = ra + ca - inter
    iou = inter / jnp.maximum(union, 1e-9)
    return (iou > _IOU_T).astype(jnp.float32)


def _nms_body(raw_s_ref, raw_t_ref, ss_ref, dets_ref, top_ref, keep_ref):
    f32 = jnp.float32
    for j in range(_NB):
        colj = _col_coords(raw_t_ref, j)
        rowj = _row_coords(raw_s_ref, j)
        supp = jnp.zeros((_B, 1), f32)
        for i in range(j):
            coli = _col_coords(raw_t_ref, i)
            s_ji = _sup_mat(rowj, coli)
            kb = keep_ref[i * _B:(i + 1) * _B, :]
            supp = supp + jax.lax.dot(s_ji, kb)
        s_dd = _sup_mat(rowj, colj)
        r_iota = jax.lax.broadcasted_iota(jnp.int32, (_B, _B), 0)
        c_iota = jax.lax.broadcasted_iota(jnp.int32, (_B, _B), 1)
        s_t = jnp.where(c_iota < r_iota, s_dd, 0.0)
        sj = ss_ref[j * _B:(j + 1) * _B, :]
        init = ((sj > _SCORE_T) & (supp < 0.5)).astype(f32)

        def cond(st):
            prev, cur, t = st
            return jnp.logical_and(t < _B + 2, jnp.any(prev != cur))

        def body(st):
            _, cur, t = st
            hits = jax.lax.dot(s_t, cur)
            new = init * (hits < 0.5).astype(f32)
            return cur, new, t + 1

        _, kj, _ = jax.lax.while_loop(
            cond, body, (init - 2.0, init, jnp.int32(0)))
        keep_ref[j * _B:(j + 1) * _B, :] = kj

    ks = keep_ref[:, :]
    blk = raw_s_ref[:, :]
    x1 = blk[:, 0:1] * 1024.0
    y1 = blk[:, 1:2] * 1024.0
    w = blk[:, 2:3] * 200.0 + 1.0
    h = blk[:, 3:4] * 200.0 + 1.0
    x2 = x1 + w
    y2 = y1 + h
    ssv = ss_ref[:, :]
    z = jnp.zeros((_NP, 1), f32)
    dets = jnp.concatenate([x1, y1, x2, y2, ssv, z, z, z], axis=1) * ks
    dets_ref[:, :] = dets

    kscore = jnp.where(ks > 0.5, ssv, -1.0)
    iota = jax.lax.broadcasted_iota(jnp.int32, (_NP, 1), 0)
    top_ref[:, :] = jnp.zeros((16, 8), f32)
    for k in range(_TOPK):
        m = jnp.max(kscore)
        pos = jnp.min(jnp.where(kscore >= m, iota, _NP))
        valid = (m > 0.0).astype(f32)
        row = dets_ref[pl.ds(pos, 1), :]
        top_ref[pl.ds(k, 1), :] = row * valid
        kscore = jnp.where(iota == pos, -1.0, kscore)


def kernel(boxes, scores):
    order = jnp.argsort(-scores)
    braw = boxes[order]
    ss = scores[order]
    braw_p = jnp.concatenate(
        [braw, jnp.zeros((_NP - _N, 4), jnp.float32)], axis=0)
    ss_p = jnp.concatenate(
        [ss, jnp.full((_NP - _N,), -1.0, jnp.float32)], axis=0)
    dets8, top8 = pl.pallas_call(
        _nms_body,
        out_shape=[
            jax.ShapeDtypeStruct((_NP, 8), jnp.float32),
            jax.ShapeDtypeStruct((16, 8), jnp.float32),
        ],
        scratch_shapes=[pltpu.VMEM((_NP, 1), jnp.float32)],
    )(braw_p, braw_p.T, ss_p[:, None])
    return dets8[:_N, :5], top8[:_TOPK, :5]


# speedup vs baseline: 146.2171x; 1.6782x over previous
import jax
import jax.numpy as jnp
from jax.experimental import pallas as pl
from jax.experimental.pallas import tpu as pltpu

_N = 5000
_B = 512
_NB = 10
_NP = _B * _NB
_M = 8192
_MR, _MC = 64, 128
_IOU_T = 0.3
_SCORE_T = 0.05
_TOPK = 15


def _bitonic_desc(key, idx, payloads):
    pos = (jax.lax.broadcasted_iota(jnp.int32, (_MR, _MC), 0) * _MC
           + jax.lax.broadcasted_iota(jnp.int32, (_MR, _MC), 1))
    kk = 2
    while kk <= _M:
        jj = kk // 2
        while jj >= 1:
            upper = (pos & jj) != 0
            if jj >= _MC:
                e = jj // _MC
                ax, s = 0, e
            else:
                ax, s = 1, jj

            def partner(a):
                return jnp.where(upper, jnp.roll(a, s, axis=ax),
                                 jnp.roll(a, -s, axis=ax))

            pk = partner(key)
            pi = partner(idx)
            desc = (pos & kk) == 0
            want_first = desc == (~upper)
            pref_other = (pk > key) | ((pk == key) & (pi < idx))
            take_other = pref_other == want_first
            key = jnp.where(take_other, pk, key)
            idx = jnp.where(take_other, pi, idx)
            payloads = [jnp.where(take_other, partner(p), p)
                        for p in payloads]
            jj //= 2
        kk *= 2
    return key, idx, payloads


def _colview(a, j):
    blk = a[4 * j:4 * (j + 1), :]
    return jnp.concatenate([blk[rr:rr + 1, :] for rr in range(4)], axis=1)


def _rowview(a, j):
    blkt = jnp.transpose(a[4 * j:4 * (j + 1), :])
    return jnp.concatenate([blkt[:, rr:rr + 1] for rr in range(4)], axis=0)


def _decode(x, y, w, h):
    x1 = x * 1024.0
    y1 = y * 1024.0
    x2 = x1 + (w * 200.0 + 1.0)
    y2 = y1 + (h * 200.0 + 1.0)
    area = (x2 - x1) * (y2 - y1)
    return x1, y1, x2, y2, area


def _row_coords(tbl_ref, j):
    blk = tbl_ref[j * _B:(j + 1) * _B, :]
    return (blk[:, 0:1], blk[:, 1:2], blk[:, 2:3], blk[:, 3:4], blk[:, 4:5])


def _col_coords(tblt_ref, j):
    blk = tblt_ref[:, j * _B:(j + 1) * _B]
    return (blk[0:1, :], blk[1:2, :], blk[2:3, :], blk[3:4, :], blk[4:5, :])


def _sup_mat(row, col):
    rx1, ry1, rx2, ry2, ra = row
    cx1, cy1, cx2, cy2, ca = col
    iw = jnp.maximum(jnp.minimum(rx2, cx2) - jnp.maximum(rx1, cx1), 0.0)
    ih = jnp.maximum(jnp.minimum(ry2, cy2) - jnp.maximum(ry1, cy1), 0.0)
    inter = iw * ih
    union = ra + ca - inter
    iou = inter / jnp.maximum(union, 1e-9)
    return (iou > _IOU_T).astype(jnp.float32)


def _nms_body(sc_ref, x_ref, y_ref, w_ref, h_ref, dets_ref, top_ref,
              tbl_ref, tblt_ref, keep_ref):
    f32 = jnp.float32
    idx0 = (jax.lax.broadcasted_iota(jnp.int32, (_MR, _MC), 0) * _MC
            + jax.lax.broadcasted_iota(jnp.int32, (_MR, _MC), 1))
    sk, _, (sx, sy, sw, sh) = _bitonic_desc(
        sc_ref[:, :], idx0,
        [x_ref[:, :], y_ref[:, :], w_ref[:, :], h_ref[:, :]])

    for j in range(_NB):
        cx1, cy1, cx2, cy2, ca = _decode(
            _colview(sx, j), _colview(sy, j), _colview(sw, j), _colview(sh, j))
        tblt_ref[0:1, j * _B:(j + 1) * _B] = cx1
        tblt_ref[1:2, j * _B:(j + 1) * _B] = cy1
        tblt_ref[2:3, j * _B:(j + 1) * _B] = cx2
        tblt_ref[3:4, j * _B:(j + 1) * _B] = cy2
        tblt_ref[4:5, j * _B:(j + 1) * _B] = ca
        rx1, ry1, rx2, ry2, ra = _decode(
            _rowview(sx, j), _rowview(sy, j), _rowview(sw, j), _rowview(sh, j))
        tbl_ref[j * _B:(j + 1) * _B, 0:1] = rx1
        tbl_ref[j * _B:(j + 1) * _B, 1:2] = ry1
        tbl_ref[j * _B:(j + 1) * _B, 2:3] = rx2
        tbl_ref[j * _B:(j + 1) * _B, 3:4] = ry2
        tbl_ref[j * _B:(j + 1) * _B, 4:5] = ra
        tbl_ref[j * _B:(j + 1) * _B, 5:6] = _rowview(sk, j)

    for j in range(_NB):
        colj = _col_coords(tblt_ref, j)
        rowj = _row_coords(tbl_ref, j)
        supp = jnp.zeros((_B, 1), f32)
        for i in range(j):
            coli = _col_coords(tblt_ref, i)
            s_ji = _sup_mat(rowj, coli)
            kb = keep_ref[i * _B:(i + 1) * _B, :]
            supp = supp + jax.lax.dot(s_ji, kb)
        s_dd = _sup_mat(rowj, colj)
        r_iota = jax.lax.broadcasted_iota(jnp.int32, (_B, _B), 0)
        c_iota = jax.lax.broadcasted_iota(jnp.int32, (_B, _B), 1)
        s_t = jnp.where(c_iota < r_iota, s_dd, 0.0)
        sj = tbl_ref[j * _B:(j + 1) * _B, 5:6]
        init = ((sj > _SCORE_T) & (supp < 0.5)).astype(f32)

        def cond(st):
            prev, cur, t = st
            return jnp.logical_and(t < _B + 2, jnp.any(prev != cur))

        def body(st):
            _, cur, t = st
            hits = jax.lax.dot(s_t, cur)
            new = init * (hits < 0.5).astype(f32)
            return cur, new, t + 1

        _, kj, _ = jax.lax.while_loop(
            cond, body, (init - 2.0, init, jnp.int32(0)))
        keep_ref[j * _B:(j + 1) * _B, :] = kj

    ks = keep_ref[:, :]
    blk = tbl_ref[:, :]
    z = jnp.zeros((_NP, 1), f32)
    dets = jnp.concatenate(
        [blk[:, 0:1], blk[:, 1:2], blk[:, 2:3], blk[:, 3:4], blk[:, 5:6],
         z, z, z], axis=1) * ks
    dets_ref[:, :] = dets

    r_iota = jax.lax.broadcasted_iota(jnp.int32, (_B, _B), 0)
    c_iota = jax.lax.broadcasted_iota(jnp.int32, (_B, _B), 1)
    ltri = (c_iota <= r_iota).astype(f32)
    kf = (jax.lax.broadcasted_iota(jnp.int32, (1, 16), 1) + 1).astype(f32)
    carry = jnp.zeros((), f32)
    acc = jnp.zeros((16, 8), f32)
    for j in range(_NB):
        kj = keep_ref[j * _B:(j + 1) * _B, :]
        rank = jax.lax.dot(ltri, kj) + carry
        carry = carry + jnp.sum(kj)
        sel = ((rank == kf) & (kj > 0.5)).astype(f32)
        dj = dets_ref[j * _B:(j + 1) * _B, :]
        acc = acc + jax.lax.dot(jnp.transpose(sel), dj,
                                precision=jax.lax.Precision.HIGHEST)
    top_ref[:, :] = acc


def kernel(boxes, scores):
    f32 = jnp.float32

    def plane(v, fill):
        return jnp.concatenate(
            [v, jnp.full((_M - _N,), fill, f32)]).reshape(_MR, _MC)

    dets8, top8 = pl.pallas_call(
        _nms_body,
        out_shape=[
            jax.ShapeDtypeStruct((_NP, 8), f32),
            jax.ShapeDtypeStruct((16, 8), f32),
        ],
        scratch_shapes=[
            pltpu.VMEM((_NP, 8), f32),
            pltpu.VMEM((8, _NP), f32),
            pltpu.VMEM((_NP, 1), f32),
        ],
    )(plane(scores, -1.0), plane(boxes[:, 0], 0.0), plane(boxes[:, 1], 0.0),
      plane(boxes[:, 2], 0.0), plane(boxes[:, 3], 0.0))
    return dets8[:_N, :5], top8[:_TOPK, :5]
